# baseline (speedup 1.0000x reference)
import jax
import jax.numpy as jnp
import numpy as np
from jax import lax
from jax.experimental import pallas as pl
from jax.experimental.pallas import tpu as pltpu
from jax.experimental.pallas import tpu_sc as plsc

_VOCAB = 1000000
_D = 64
_NEG = 10
_B = 4096
_W = 50

_NC, _NS = 2, 16
_NW = _NC * _NS
_NB = _B // _NW
_CI = 16
_NCH = _NB // _CI
_TN = _NEG + 1

_CTX_DMA = 100
_CTX_NDMA = _CI * _W // _CTX_DMA
_TN_DMA = 128
_TN_NDMA = _NB * _TN // _TN_DMA

_F16_SCALE = np.float32(5.192296858534828e33)


def _halves_to_f32(x_i32):
    w = lax.bitcast_convert_type(x_i32, jnp.uint32)
    lo = w << np.uint32(16)
    hi = w & np.uint32(0xFFFF0000)

    def one(s):
        sign = s & np.uint32(0x80000000)
        mag = (s >> np.uint32(3)) & np.uint32(0x0FFFE000)
        return lax.bitcast_convert_type(sign | mag, jnp.float32) * _F16_SCALE

    return one(lo), one(hi)


def _row_to_f32(rows_ref, r):
    a, b = _halves_to_f32(rows_ref[r, pl.ds(0, 16)])
    c, d = _halves_to_f32(rows_ref[r, pl.ds(16, 16)])
    return a, b, c, d


def _sc_body(ctx_tab, tgt_tab, ctx_idx_h, tn_idx_h, dots_h,
             ctx_idx_v, tn_idx_v, ctx_rows, tn_rows, dots_v, sem, tn_sem):
    wid = lax.axis_index("s") * _NC + lax.axis_index("c")
    pltpu.sync_copy(ctx_idx_h.at[wid], ctx_idx_v)
    pltpu.sync_copy(tn_idx_h.at[wid], tn_idx_v)
    inv_w = np.float32(1.0 / _W)

    tn_cps = []
    for j in range(_TN_NDMA):
        tn_cps.append(pltpu.async_copy(
            tgt_tab.at[tn_idx_v.at[j]],
            tn_rows.at[pl.ds(j * _TN_DMA, _TN_DMA)], tn_sem))

    for c in range(_NCH):
        cps = []
        for j in range(_CTX_NDMA):
            cps.append(pltpu.async_copy(
                ctx_tab.at[ctx_idx_v.at[c * _CTX_NDMA + j]],
                ctx_rows.at[pl.ds(j * _CTX_DMA, _CTX_DMA)], sem))
        for cp in cps:
            cp.wait()
        if c == 0:
            for cp in tn_cps:
                cp.wait()

        def item_body(i, carry):
            def row_body(r, acc):
                a0, a1, a2, a3 = acc
                b0, b1, b2, b3 = _row_to_f32(ctx_rows, i * _W + r)
                return (a0 + b0, a1 + b1, a2 + b2, a3 + b3)

            z = jnp.zeros((16,), jnp.float32)
            s0, s1, s2, s3 = lax.fori_loop(0, _W, row_body, (z, z, z, z))
            c0 = s0 * inv_w
            c1 = s1 * inv_w
            c2 = s2 * inv_w
            c3 = s3 * inv_w
            base = (c * _CI + i) * _TN
            for k in range(_TN):
                t0, t1, t2, t3 = _row_to_f32(tn_rows, base + k)
                dots_v[i * _TN + k, :] = (
                    t0 * c0 + t1 * c1 + t2 * c2 + t3 * c3)
            return carry

        lax.fori_loop(0, _CI, item_body, 0)
        pltpu.sync_copy(
            dots_v,
            dots_h.at[pl.ds((wid * _NB + c * _CI) * _TN, _CI * _TN)])


def _loss_body(x_ref, o_ref):
    x = x_ref[...]
    d = jnp.sum(x, axis=1)
    ls = jnp.log(jax.nn.sigmoid(d))
    o_ref[...] = jnp.reshape(-jnp.sum(ls) / np.float32(_B), (1, 1))


def kernel(context, target, emb, ctx_emb):
    noise = jax.random.randint(
        jax.random.key(1), (_B, _NEG), 0, _VOCAB)
    ctx_idx = context.astype(jnp.int32).reshape(
        _NW, _NB * _W // _CTX_DMA, _CTX_DMA)
    tn_idx = jnp.concatenate(
        [target.astype(jnp.int32)[:, None], noise.astype(jnp.int32)],
        axis=1).reshape(_NW, _TN_NDMA, _TN_DMA)
    ctx_tab = lax.bitcast_convert_type(
        ctx_emb.reshape(_VOCAB, _D // 2, 2), jnp.int32)
    tgt_tab = lax.bitcast_convert_type(
        emb.reshape(_VOCAB, _D // 2, 2), jnp.int32)

    mesh = plsc.VectorSubcoreMesh(
        core_axis_name="c", subcore_axis_name="s",
        num_cores=_NC, num_subcores=_NS)
    sc = pl.kernel(
        _sc_body,
        out_type=jax.ShapeDtypeStruct((_B * _TN, 16), jnp.float32),
        mesh=mesh,
        scratch_types=[
            pltpu.VMEM((_NB * _W // _CTX_DMA, _CTX_DMA), jnp.int32),
            pltpu.VMEM((_TN_NDMA, _TN_DMA), jnp.int32),
            pltpu.VMEM((_CI * _W, _D // 2), jnp.int32),
            pltpu.VMEM((_NB * _TN, _D // 2), jnp.int32),
            pltpu.VMEM((_CI * _TN, 16), jnp.float32),
            pltpu.SemaphoreType.DMA,
            pltpu.SemaphoreType.DMA,
        ],
        compiler_params=pltpu.CompilerParams(use_tc_tiling_on_sc=False),
    )
    dots = sc(ctx_tab, tgt_tab, ctx_idx, tn_idx)

    loss2d = pl.pallas_call(
        _loss_body,
        out_shape=jax.ShapeDtypeStruct((1, 1), jnp.float32),
    )(dots)
    return loss2d[0, 0].astype(jnp.float16)


# speedup vs baseline: 2.1833x; 2.1833x over previous
import jax
import jax.numpy as jnp
import numpy as np
from jax import lax
from jax.experimental import pallas as pl
from jax.experimental.pallas import tpu as pltpu
from jax.experimental.pallas import tpu_sc as plsc

_VOCAB = 1000000
_D = 64
_NEG = 10
_B = 4096
_W = 50

_NC, _NS = 2, 16
_NW = _NC * _NS
_NB = _B // _NW
_CI = 16
_NCH = _NB // _CI
_TN = _NEG + 1

_CTX_DMA = 100
_CTX_NDMA = _CI * _W // _CTX_DMA
_TN_DMA = 128
_TN_NDMA = _NB * _TN // _TN_DMA

_F16_SCALE = np.float32(5.192296858534828e33)


def _f16_bits_to_f32(h):
    sign = (h & np.uint32(0x8000)) << np.uint32(16)
    mag = (h & np.uint32(0x7FFF)) << np.uint32(13)
    return lax.bitcast_convert_type(sign | mag, jnp.float32) * _F16_SCALE


def _halves_to_f32(x_u16):
    a, b = plsc.unpack(x_u16, format=plsc.PackFormat.INTERLEAVED,
                       preferred_element_type=jnp.uint32)
    return _f16_bits_to_f32(a), _f16_bits_to_f32(b)


def _row_to_f32(rows_ref, r):
    a, b = _halves_to_f32(rows_ref[r, pl.ds(0, 32)])
    c, d = _halves_to_f32(rows_ref[r, pl.ds(32, 32)])
    return a, b, c, d


def _sc_body(ctx_tab, tgt_tab, ctx_idx_h, tn_idx_h, dots_h,
             ctx_idx_v, tn_idx_v, ctx_rows, tn_rows, dots_v, sem, tn_sem):
    wid = lax.axis_index("s") * _NC + lax.axis_index("c")
    pltpu.sync_copy(ctx_idx_h.at[wid], ctx_idx_v)
    pltpu.sync_copy(tn_idx_h.at[wid], tn_idx_v)
    inv_w = np.float32(1.0 / _W)

    tn_cps = []
    for j in range(_TN_NDMA):
        tn_cps.append(pltpu.async_copy(
            tgt_tab.at[tn_idx_v.at[j]],
            tn_rows.at[pl.ds(j * _TN_DMA, _TN_DMA)], tn_sem))

    for c in range(_NCH):
        cps = []
        for j in range(_CTX_NDMA):
            cps.append(pltpu.async_copy(
                ctx_tab.at[ctx_idx_v.at[c * _CTX_NDMA + j]],
                ctx_rows.at[pl.ds(j * _CTX_DMA, _CTX_DMA)], sem))
        for cp in cps:
            cp.wait()
        if c == 0:
            for cp in tn_cps:
                cp.wait()

        def item_body(i, carry):
            def row_body(r, acc):
                a0, a1, a2, a3 = acc
                b0, b1, b2, b3 = _row_to_f32(ctx_rows, i * _W + r)
                return (a0 + b0, a1 + b1, a2 + b2, a3 + b3)

            z = jnp.zeros((16,), jnp.float32)
            s0, s1, s2, s3 = lax.fori_loop(0, _W, row_body, (z, z, z, z))
            c0 = s0 * inv_w
            c1 = s1 * inv_w
            c2 = s2 * inv_w
            c3 = s3 * inv_w
            base = (c * _CI + i) * _TN
            for k in range(_TN):
                t0, t1, t2, t3 = _row_to_f32(tn_rows, base + k)
                dots_v[i * _TN + k, :] = (
                    t0 * c0 + t1 * c1 + t2 * c2 + t3 * c3)
            return carry

        lax.fori_loop(0, _CI, item_body, 0)
        pltpu.sync_copy(
            dots_v,
            dots_h.at[pl.ds((wid * _NB + c * _CI) * _TN, _CI * _TN)])


def _loss_body(x_ref, o_ref):
    x = x_ref[...]
    d = jnp.sum(x, axis=1)
    ls = jnp.log(jax.nn.sigmoid(d))
    o_ref[...] = jnp.reshape(-jnp.sum(ls) / np.float32(_B), (1, 1))


def kernel(context, target, emb, ctx_emb):
    noise = jax.random.randint(
        jax.random.key(1), (_B, _NEG), 0, _VOCAB)
    ctx_idx = context.astype(jnp.int32).reshape(
        _NW, _NB * _W // _CTX_DMA, _CTX_DMA)
    tn_idx = jnp.concatenate(
        [target.astype(jnp.int32)[:, None], noise.astype(jnp.int32)],
        axis=1).reshape(_NW, _TN_NDMA, _TN_DMA)
    ctx_tab = lax.bitcast_convert_type(ctx_emb, jnp.uint16)
    tgt_tab = lax.bitcast_convert_type(emb, jnp.uint16)

    mesh = plsc.VectorSubcoreMesh(
        core_axis_name="c", subcore_axis_name="s",
        num_cores=_NC, num_subcores=_NS)
    sc = pl.kernel(
        _sc_body,
        out_type=jax.ShapeDtypeStruct((_B * _TN, 16), jnp.float32),
        mesh=mesh,
        scratch_types=[
            pltpu.VMEM((_NB * _W // _CTX_DMA, _CTX_DMA), jnp.int32),
            pltpu.VMEM((_TN_NDMA, _TN_DMA), jnp.int32),
            pltpu.VMEM((_CI * _W, _D), jnp.uint16),
            pltpu.VMEM((_NB * _TN, _D), jnp.uint16),
            pltpu.VMEM((_CI * _TN, 16), jnp.float32),
            pltpu.SemaphoreType.DMA,
            pltpu.SemaphoreType.DMA,
        ],
        compiler_params=pltpu.CompilerParams(
            use_tc_tiling_on_sc=False, needs_layout_passes=False),
    )
    dots = sc(ctx_tab, tgt_tab, ctx_idx, tn_idx)

    loss2d = pl.pallas_call(
        _loss_body,
        out_shape=jax.ShapeDtypeStruct((1, 1), jnp.float32),
    )(dots)
    return loss2d[0, 0].astype(jnp.float16)


# speedup vs baseline: 2.8988x; 1.3277x over previous
import jax
import jax.numpy as jnp
import numpy as np
from jax import lax
from jax.experimental import pallas as pl
from jax.experimental.pallas import tpu as pltpu
from jax.experimental.pallas import tpu_sc as plsc

_VOCAB = 1000000
_D = 64
_NEG = 10
_B = 4096
_W = 50

_NC, _NS = 2, 16
_NW = _NC * _NS
_NB = _B // _NW
_CI = 8
_NCH = _NB // _CI
_TN = _NEG + 1

_CTX_DMA = 100
_CTX_NDMA = _CI * _W // _CTX_DMA
_TN_DMA = _CI * _TN

_F16_SCALE = np.float32(5.192296858534828e33)


def _f16_bits_to_f32(h):
    sign = (h & np.uint32(0x8000)) << np.uint32(16)
    mag = (h & np.uint32(0x7FFF)) << np.uint32(13)
    return lax.bitcast_convert_type(sign | mag, jnp.float32) * _F16_SCALE


def _fetch_row(rows_ref, j, q, cb):
    out = []
    for k in range(4):
        w = lax.bitcast_convert_type(
            rows_ref[j, pl.ds(cb + 16 * k, 16)], jnp.uint32)
        if q:
            w = w >> np.uint32(16)
        out.append(_f16_bits_to_f32(w))
    return out


def _sc_body(ctx_tab, tgt_tab, ctx_idx_h, tn_idx_h, bnd_h, dots_h,
             ctx_idx_v, tn_idx_v, bnd_v, ctx_rows, tn_rows, dots_v,
             sem, tn_sem):
    wid = lax.axis_index("s") * _NC + lax.axis_index("c")
    pltpu.sync_copy(ctx_idx_h.at[wid], ctx_idx_v)
    pltpu.sync_copy(tn_idx_h.at[wid], tn_idx_v)
    pltpu.sync_copy(bnd_h.at[wid], bnd_v)
    ctx32 = ctx_tab.bitcast(jnp.int32)
    tgt32 = tgt_tab.bitcast(jnp.int32)
    inv_w = np.float32(1.0 / _W)

    def chunk_body(c, carry):
        cps = []
        for j in range(_CTX_NDMA):
            cps.append(pltpu.async_copy(
                ctx32.at[ctx_idx_v.at[c * _CTX_NDMA + j]],
                ctx_rows.at[pl.ds(j * _CTX_DMA, _CTX_DMA)], sem))
        cps.append(pltpu.async_copy(
            tgt32.at[tn_idx_v.at[c]], tn_rows, tn_sem))
        for cp in cps:
            cp.wait()

        def item_body(i, carry2):
            gl = c * _CI + i
            b = bnd_v[pl.ds(gl * 8, 16)]
            ce = [0, b[0], b[1], b[2], _W]
            te = [0, b[3], b[4], b[5], _TN]
            acc = [jnp.zeros((16,), jnp.float32) for _ in range(4)]
            for m in range(4):
                q, cb = m // 2, (m & 1) * 64

                def ctx_sub(r, a):
                    f = _fetch_row(ctx_rows, i * _W + r, q, cb)
                    return (a[0] + f[0], a[1] + f[1],
                            a[2] + f[2], a[3] + f[3])

                acc = list(lax.fori_loop(ce[m], ce[m + 1], ctx_sub,
                                         tuple(acc)))
            cv = [a * inv_w for a in acc]
            for m in range(4):
                q, cb = m // 2, (m & 1) * 64

                def tn_sub(r, carry3):
                    t = _fetch_row(tn_rows, i * _TN + r, q, cb)
                    dots_v[i * _TN + r, :] = (
                        t[0] * cv[0] + t[1] * cv[1]
                        + t[2] * cv[2] + t[3] * cv[3])
                    return carry3

                lax.fori_loop(te[m], te[m + 1], tn_sub, 0)
            return carry2

        lax.fori_loop(0, _CI, item_body, 0)
        pltpu.sync_copy(
            dots_v,
            dots_h.at[pl.ds((wid * _NB + c * _CI) * _TN, _CI * _TN)])
        return carry

    lax.fori_loop(0, _NCH, chunk_body, 0)


def _loss_body(x_ref, o_ref):
    x = x_ref[...]
    d = jnp.sum(x, axis=1)
    ls = jnp.log(jax.nn.sigmoid(d))
    o_ref[...] = jnp.reshape(-jnp.sum(ls) / np.float32(_B), (1, 1))


def _sort_by_residue(idx, width):
    key = ((idx & 3) << 20) + idx
    skey = jnp.sort(key, axis=1)
    v_sorted = skey & ((1 << 20) - 1)
    res = skey >> 20
    bnds = [jnp.sum((res <= m).astype(jnp.int32), axis=1)
            for m in range(3)]
    return v_sorted >> 2, jnp.stack(bnds, axis=1)


def kernel(context, target, emb, ctx_emb):
    noise = jax.random.randint(
        jax.random.key(1), (_B, _NEG), 0, _VOCAB)
    tn_all = jnp.concatenate(
        [target.astype(jnp.int32)[:, None], noise.astype(jnp.int32)],
        axis=1)
    ctx_g, ctx_b = _sort_by_residue(context.astype(jnp.int32), _W)
    tn_g, tn_b = _sort_by_residue(tn_all, _TN)

    ctx_idx = ctx_g.reshape(_NW, _NB * _W // _CTX_DMA, _CTX_DMA)
    tn_idx = tn_g.reshape(_NW, _NCH, _TN_DMA)
    bounds = jnp.concatenate(
        [ctx_b, tn_b, jnp.zeros((_B, 2), jnp.int32)], axis=1)
    bounds = jnp.pad(bounds.reshape(_NW, _NB * 8), ((0, 0), (0, 8)))

    ctx_tab = lax.bitcast_convert_type(
        ctx_emb, jnp.uint16).reshape(_VOCAB // 2, 128)
    tgt_tab = lax.bitcast_convert_type(
        emb, jnp.uint16).reshape(_VOCAB // 2, 128)

    mesh = plsc.VectorSubcoreMesh(
        core_axis_name="c", subcore_axis_name="s",
        num_cores=_NC, num_subcores=_NS)
    sc = pl.kernel(
        _sc_body,
        out_type=jax.ShapeDtypeStruct((_B * _TN, 16), jnp.float32),
        mesh=mesh,
        scratch_types=[
            pltpu.VMEM((_NB * _W // _CTX_DMA, _CTX_DMA), jnp.int32),
            pltpu.VMEM((_NCH, _TN_DMA), jnp.int32),
            pltpu.VMEM((_NB * 8 + 8,), jnp.int32),
            pltpu.VMEM((_CI * _W, 128), jnp.int32),
            pltpu.VMEM((_CI * _TN, 128), jnp.int32),
            pltpu.VMEM((_CI * _TN, 16), jnp.float32),
            pltpu.SemaphoreType.DMA,
            pltpu.SemaphoreType.DMA,
        ],
        compiler_params=pltpu.CompilerParams(needs_layout_passes=False),
    )
    dots = sc(ctx_tab, tgt_tab, ctx_idx, tn_idx, bounds)

    loss2d = pl.pallas_call(
        _loss_body,
        out_shape=jax.ShapeDtypeStruct((1, 1), jnp.float32),
    )(dots)
    return loss2d[0, 0].astype(jnp.float16)


# speedup vs baseline: 3.2964x; 1.1372x over previous
import jax
import jax.numpy as jnp
import numpy as np
from jax import lax
from jax.experimental import pallas as pl
from jax.experimental.pallas import tpu as pltpu
from jax.experimental.pallas import tpu_sc as plsc

_VOCAB = 1000000
_D = 64
_NEG = 10
_B = 4096
_W = 50

_NC, _NS = 2, 16
_NW = _NC * _NS
_NB = _B // _NW
_CI = 8
_NCH = _NB // _CI
_TN = _NEG + 1

_CTX_DMA = 100
_CTX_NDMA = _CI * _W // _CTX_DMA
_TN_DMA = _CI * _TN

_F16_SCALE = np.float32(5.192296858534828e33)


def _f16_bits_to_f32(h):
    sign = (h & np.uint32(0x8000)) << np.uint32(16)
    mag = (h & np.uint32(0x7FFF)) << np.uint32(13)
    return lax.bitcast_convert_type(sign | mag, jnp.float32) * _F16_SCALE


def _fetch_row(rows_ref, j, q, cb):
    out = []
    for k in range(4):
        w = lax.bitcast_convert_type(
            rows_ref[j, pl.ds(cb + 16 * k, 16)], jnp.uint32)
        if q:
            w = w >> np.uint32(16)
        out.append(_f16_bits_to_f32(w))
    return out


def _sc_body(ctx_tab, tgt_tab, ctx_idx_h, tn_idx_h, bnd_h, dots_h,
             ctx_idx_v, tn_idx_v, bnd_v, ctx_rows, tn_rows, dots_v,
             sem, tn_sem):
    wid = lax.axis_index("s") * _NC + lax.axis_index("c")
    pltpu.sync_copy(ctx_idx_h.at[wid], ctx_idx_v)
    pltpu.sync_copy(tn_idx_h.at[wid], tn_idx_v)
    pltpu.sync_copy(bnd_h.at[wid], bnd_v)
    ctx32 = ctx_tab.bitcast(jnp.int32)
    tgt32 = tgt_tab.bitcast(jnp.int32)
    inv_w = np.float32(1.0 / _W)

    def chunk_body(c, carry):
        cps = []
        for j in range(_CTX_NDMA):
            cps.append(pltpu.async_copy(
                ctx32.at[ctx_idx_v.at[c * _CTX_NDMA + j]],
                ctx_rows.at[pl.ds(j * _CTX_DMA, _CTX_DMA)], sem))
        cps.append(pltpu.async_copy(
            tgt32.at[tn_idx_v.at[c]], tn_rows, tn_sem))
        for cp in cps:
            cp.wait()

        def item_body(i, carry2):
            gl = c * _CI + i
            b = bnd_v[pl.ds(gl * 8, 16)]
            ce = [0, b[0], b[1], b[2], _W]
            te = [0, b[3], b[4], b[5], _TN]
            acc = [jnp.zeros((16,), jnp.float32) for _ in range(4)]
            for m in range(4):
                q, cb = m // 2, (m & 1) * 64

                def ctx_sub(r, a):
                    f = _fetch_row(ctx_rows, i * _W + r, q, cb)
                    return (a[0] + f[0], a[1] + f[1],
                            a[2] + f[2], a[3] + f[3])

                acc = list(lax.fori_loop(ce[m], ce[m + 1], ctx_sub,
                                         tuple(acc)))
            cv = [a * inv_w for a in acc]
            for m in range(4):
                q, cb = m // 2, (m & 1) * 64

                def tn_sub(r, carry3):
                    t = _fetch_row(tn_rows, i * _TN + r, q, cb)
                    dots_v[i * _TN + r, :] = (
                        t[0] * cv[0] + t[1] * cv[1]
                        + t[2] * cv[2] + t[3] * cv[3])
                    return carry3

                lax.fori_loop(te[m], te[m + 1], tn_sub, 0)
            return carry2

        lax.fori_loop(0, _CI, item_body, 0)
        pltpu.sync_copy(
            dots_v,
            dots_h.at[pl.ds((wid * _NB + c * _CI) * _TN, _CI * _TN)])
        return carry

    lax.fori_loop(0, _NCH, chunk_body, 0)


def _loss_body(x_ref, o_ref):
    x = x_ref[...]
    d = jnp.sum(x, axis=1)
    ls = jnp.log(jax.nn.sigmoid(d))
    o_ref[...] = jnp.reshape(-jnp.sum(ls) / np.float32(_B), (1, 1))


def _sort_by_residue(idx, width):
    key = ((idx & 3) << 20) + idx
    skey = jnp.sort(key, axis=1)
    v_sorted = skey & ((1 << 20) - 1)
    res = skey >> 20
    bnds = [jnp.sum((res <= m).astype(jnp.int32), axis=1)
            for m in range(3)]
    return v_sorted >> 2, jnp.stack(bnds, axis=1)


def kernel(context, target, emb, ctx_emb):
    noise = jax.random.randint(
        jax.random.key(1), (_B, _NEG), 0, _VOCAB)
    tn_all = jnp.concatenate(
        [target.astype(jnp.int32)[:, None], noise.astype(jnp.int32)],
        axis=1)
    ctx_g, ctx_b = _sort_by_residue(context.astype(jnp.int32), _W)
    tn_g, tn_b = _sort_by_residue(tn_all, _TN)

    ctx_idx = ctx_g.reshape(_NW, _NB * _W // _CTX_DMA, _CTX_DMA)
    tn_idx = tn_g.reshape(_NW, _NCH, _TN_DMA)
    bounds = jnp.concatenate(
        [ctx_b, tn_b, jnp.zeros((_B, 2), jnp.int32)], axis=1)
    bounds = jnp.pad(bounds.reshape(_NW, _NB * 8), ((0, 0), (0, 8)))

    ctx_tab = ctx_emb.reshape(_VOCAB // 2, 128)
    tgt_tab = emb.reshape(_VOCAB // 2, 128)

    mesh = plsc.VectorSubcoreMesh(
        core_axis_name="c", subcore_axis_name="s",
        num_cores=_NC, num_subcores=_NS)
    sc = pl.kernel(
        _sc_body,
        out_type=jax.ShapeDtypeStruct((_B * _TN, 16), jnp.float32),
        mesh=mesh,
        scratch_types=[
            pltpu.VMEM((_NB * _W // _CTX_DMA, _CTX_DMA), jnp.int32),
            pltpu.VMEM((_NCH, _TN_DMA), jnp.int32),
            pltpu.VMEM((_NB * 8 + 8,), jnp.int32),
            pltpu.VMEM((_CI * _W, 128), jnp.int32),
            pltpu.VMEM((_CI * _TN, 128), jnp.int32),
            pltpu.VMEM((_CI * _TN, 16), jnp.float32),
            pltpu.SemaphoreType.DMA,
            pltpu.SemaphoreType.DMA,
        ],
        compiler_params=pltpu.CompilerParams(needs_layout_passes=False),
    )
    dots = sc(ctx_tab, tgt_tab, ctx_idx, tn_idx, bounds)

    loss2d = pl.pallas_call(
        _loss_body,
        out_shape=jax.ShapeDtypeStruct((1, 1), jnp.float32),
    )(dots)
    return loss2d[0, 0].astype(jnp.float16)
